# baseline (speedup 1.0000x reference)
import functools

import jax
import jax.numpy as jnp
from jax import lax
from jax.experimental import pallas as pl
from jax.experimental.pallas import tpu as pltpu
from jax.experimental.pallas import tpu_sc as plsc

EMBED = 768
HIDDEN = 3072
E = 8
T = 2048
BLK = 512
PP = T * 2 + E * BLK
NB = PP // BLK
CB = 128
NCB = T // CB

NC, NS = 2, 16
NW = NC * NS
TOK_W = T // NW
GW = 128


def _router_body(x_ref, wg_ref, d0_ref, d1_ref, w0e_ref, w1e_ref, cnt_ref):
    logits = lax.dot_general(x_ref[...], wg_ref[...], (((1,), (1,)), ((), ())),
                             preferred_element_type=jnp.float32)
    m = jnp.max(logits, axis=1, keepdims=True)
    p = jnp.exp(logits - m)
    p = p / jnp.sum(p, axis=1, keepdims=True)
    iot = lax.broadcasted_iota(jnp.int32, (T, E), 1)
    m1 = jnp.max(p, axis=1, keepdims=True)
    i1 = jnp.min(jnp.where(p == m1, iot, E), axis=1, keepdims=True)
    sel1 = (iot == i1).astype(jnp.float32)
    p2 = jnp.where(iot == i1, -1.0, p)
    m2 = jnp.max(p2, axis=1, keepdims=True)
    i2 = jnp.min(jnp.where(p2 == m2, iot, E), axis=1, keepdims=True)
    sel2 = (iot == i2).astype(jnp.float32)
    den = m1 + m2 + 1e-9
    w0e_ref[...] = jnp.broadcast_to(m1 / den, (T, GW))
    w1e_ref[...] = jnp.broadcast_to(m2 / den, (T, GW))

    oh = sel1 + sel2
    counts = jnp.sum(oh, axis=0, keepdims=True)
    nblk = jnp.floor((counts + (BLK - 1)) * (1.0 / BLK))
    eiota = lax.broadcasted_iota(jnp.int32, (E, E), 0)
    ejota = lax.broadcasted_iota(jnp.int32, (E, E), 1)
    upper = (eiota < ejota).astype(jnp.float32)
    blk_start = lax.dot_general(nblk, upper, (((1,), (0,)), ((), ())),
                                preferred_element_type=jnp.float32)
    off = blk_start * BLK
    oh3 = oh.reshape(NCB, CB, E)
    riota = lax.broadcasted_iota(jnp.int32, (CB, CB), 0)
    ciota = lax.broadcasted_iota(jnp.int32, (CB, CB), 1)
    ltri = (ciota < riota).astype(jnp.float32)
    local = jax.vmap(
        lambda blkoh: lax.dot_general(ltri, blkoh, (((1,), (0,)), ((), ())),
                                      preferred_element_type=jnp.float32)
    )(oh3)
    bsum = jnp.sum(oh3, axis=1)
    carry = bsum
    sh = 1
    while sh < NCB:
        carry = carry + jnp.concatenate(
            [jnp.zeros((sh, E), jnp.float32), carry[:NCB - sh]], axis=0)
        sh *= 2
    carry = carry - bsum
    cum = (local + carry[:, None, :]).reshape(T, E)
    rank0 = jnp.sum(cum * sel1, axis=1, keepdims=True)
    rank1 = jnp.sum(cum * sel2, axis=1, keepdims=True)
    dst0 = jnp.sum(off * sel1, axis=1, keepdims=True) + rank0
    dst1 = jnp.sum(off * sel2, axis=1, keepdims=True) + rank1
    d0_ref[...] = dst0.astype(jnp.int32)
    d1_ref[...] = dst1.astype(jnp.int32)
    cnt_ref[...] = counts


def _router(xf, Wg):
    return pl.pallas_call(
        _router_body,
        out_shape=(
            jax.ShapeDtypeStruct((T, 1), jnp.int32),
            jax.ShapeDtypeStruct((T, 1), jnp.int32),
            jax.ShapeDtypeStruct((T, GW), jnp.float32),
            jax.ShapeDtypeStruct((T, GW), jnp.float32),
            jax.ShapeDtypeStruct((1, E), jnp.float32),
        ),
    )(xf, Wg)


def _dispatch_body(x_hbm, d0_hbm, d1_hbm, w0e_hbm, w1e_hbm, xs_hbm, gsl_hbm,
                   i0_v, i1_v, w0_v, w1_v, rows_v, ssem):
    wid = lax.axis_index("s") * NC + lax.axis_index("c")
    base = wid * TOK_W
    pltpu.sync_copy(d0_hbm.at[pl.ds(base, TOK_W)], i0_v)
    pltpu.sync_copy(d1_hbm.at[pl.ds(base, TOK_W)], i1_v)
    pltpu.sync_copy(w0e_hbm.at[pl.ds(base, TOK_W)], w0_v)
    pltpu.sync_copy(w1e_hbm.at[pl.ds(base, TOK_W)], w1_v)
    pltpu.sync_copy(x_hbm.at[pl.ds(base, TOK_W)], rows_v)
    ops = [
        pltpu.async_copy(rows_v, xs_hbm.at[i0_v], ssem),
        pltpu.async_copy(rows_v, xs_hbm.at[i1_v], ssem),
        pltpu.async_copy(w0_v, gsl_hbm.at[i0_v], ssem),
        pltpu.async_copy(w1_v, gsl_hbm.at[i1_v], ssem),
    ]
    for s in ops:
        s.wait()


def _dispatch(xf, dst0, dst1, w0e, w1e):
    mesh = plsc.VectorSubcoreMesh(core_axis_name="c", subcore_axis_name="s")
    fn = functools.partial(
        pl.kernel, mesh=mesh,
        out_type=(
            jax.ShapeDtypeStruct((PP, EMBED), jnp.float32),
            jax.ShapeDtypeStruct((PP, GW), jnp.float32),
        ),
        scratch_types=[
            pltpu.VMEM((TOK_W,), jnp.int32),
            pltpu.VMEM((TOK_W,), jnp.int32),
            pltpu.VMEM((TOK_W, GW), jnp.float32),
            pltpu.VMEM((TOK_W, GW), jnp.float32),
            pltpu.VMEM((TOK_W, EMBED), jnp.float32),
            pltpu.SemaphoreType.DMA,
        ],
    )(_dispatch_body)
    return fn(xf, dst0, dst1, w0e, w1e)


def _ffn_body(be_ref, xs_ref, gsl_ref, w1_ref, b1_ref, w2_ref, b2_ref, out_ref):
    i = pl.program_id(0)

    @pl.when(i < be_ref[NB])
    def _():
        h = lax.dot_general(xs_ref[...], w1_ref[0], (((1,), (1,)), ((), ())),
                            preferred_element_type=jnp.float32)
        h = jnp.maximum(h + b1_ref[0, 0][None, :], 0.0)
        y = lax.dot_general(h, w2_ref[0], (((1,), (1,)), ((), ())),
                            preferred_element_type=jnp.float32)
        g = gsl_ref[...][:, 0:1]
        out_ref[...] = (y + b2_ref[0, 0][None, :]) * g


def _ffn(be, xs, gsl, W1, b1r, W2, b2r):
    grid_spec = pltpu.PrefetchScalarGridSpec(
        num_scalar_prefetch=1,
        grid=(NB,),
        in_specs=[
            pl.BlockSpec((BLK, EMBED), lambda i, be: (i, 0)),
            pl.BlockSpec((BLK, GW), lambda i, be: (i, 0)),
            pl.BlockSpec((1, HIDDEN, EMBED), lambda i, be: (be[i], 0, 0)),
            pl.BlockSpec((1, 1, HIDDEN), lambda i, be: (be[i], 0, 0)),
            pl.BlockSpec((1, EMBED, HIDDEN), lambda i, be: (be[i], 0, 0)),
            pl.BlockSpec((1, 1, EMBED), lambda i, be: (be[i], 0, 0)),
        ],
        out_specs=pl.BlockSpec((BLK, EMBED), lambda i, be: (i, 0)),
    )
    return pl.pallas_call(
        _ffn_body,
        grid_spec=grid_spec,
        out_shape=jax.ShapeDtypeStruct((PP, EMBED), jnp.float32),
        compiler_params=pltpu.CompilerParams(
            dimension_semantics=("arbitrary",)),
    )(be, xs, gsl, W1, b1r, W2, b2r)


def _combine_body(ys_hbm, dst0_hbm, dst1_hbm, out_hbm,
                  i0_v, i1_v, a_v, b_v, sem):
    wid = lax.axis_index("s") * NC + lax.axis_index("c")
    base = wid * TOK_W
    pltpu.sync_copy(dst0_hbm.at[pl.ds(base, TOK_W)], i0_v)
    pltpu.sync_copy(dst1_hbm.at[pl.ds(base, TOK_W)], i1_v)
    c0 = pltpu.async_copy(ys_hbm.at[i0_v], a_v, sem)
    c1 = pltpu.async_copy(ys_hbm.at[i1_v], b_v, sem)
    c0.wait()
    c1.wait()

    def body(t, carry):
        for c in range(EMBED // 16):
            sl = pl.ds(c * 16, 16)
            a_v[t, sl] = a_v[t, sl] + b_v[t, sl]
        return carry

    lax.fori_loop(0, TOK_W, body, 0)
    pltpu.sync_copy(a_v, out_hbm.at[pl.ds(base, TOK_W)])


def _combine(ys, dst0, dst1):
    mesh = plsc.VectorSubcoreMesh(core_axis_name="c", subcore_axis_name="s")
    fn = functools.partial(
        pl.kernel, mesh=mesh,
        out_type=jax.ShapeDtypeStruct((T, EMBED), jnp.float32),
        scratch_types=[
            pltpu.VMEM((TOK_W,), jnp.int32),
            pltpu.VMEM((TOK_W,), jnp.int32),
            pltpu.VMEM((TOK_W, EMBED), jnp.float32),
            pltpu.VMEM((TOK_W, EMBED), jnp.float32),
            pltpu.SemaphoreType.DMA,
        ],
    )(_combine_body)
    return fn(ys, dst0, dst1)


def kernel(x, Wg, W1, b1, W2, b2):
    orig_shape = x.shape
    xf = x.reshape(T, EMBED)
    d0c, d1c, w0e, w1e, counts = _router(xf, Wg)
    dst0 = d0c.reshape(T)
    dst1 = d1c.reshape(T)
    nblk = jnp.ceil(counts[0] * (1.0 / BLK)).astype(jnp.int32)
    blk_start = jnp.concatenate(
        [jnp.zeros((1,), jnp.int32), jnp.cumsum(nblk)[:-1].astype(jnp.int32)])
    bidx = jnp.arange(NB, dtype=jnp.int32)
    be = jnp.clip(jnp.sum((bidx[:, None] >= blk_start[None, :]).astype(jnp.int32),
                          axis=1) - 1, 0, E - 1).astype(jnp.int32)
    be = jnp.concatenate([be, jnp.sum(nblk, keepdims=True)])
    xs, gsl = _dispatch(xf, dst0, dst1, w0e, w1e)
    b1r = b1.reshape(E, 1, HIDDEN)
    b2r = b2.reshape(E, 1, EMBED)
    ys = _ffn(be, xs, gsl, W1, b1r, W2, b2r)
    out = _combine(ys, dst0, dst1)
    return out.reshape(orig_shape)
